# Initial kernel scaffold; baseline (speedup 1.0000x reference)
import jax
import jax.numpy as jnp
from jax.experimental import pallas as pl


def kernel(objectness, box_regression, anchors):
    raise NotImplementedError("write your pallas kernel here")


# speedup vs baseline: 30.3539x; 30.3539x over previous
import functools

import jax
import jax.numpy as jnp
import numpy as np
from jax.experimental import pallas as pl
from jax.experimental.pallas import tpu as pltpu

_PRE = 2000
_POST = 1000
_THRESH = 0.7
_IMG_W = 800.0
_IMG_H = 800.0
_CLIP = float(np.log(1000.0 / 16.0))


def _decode_clip(dx, dy, dw, dh, ax1, ay1, ax2, ay2):
    dw = jnp.minimum(dw, _CLIP)
    dh = jnp.minimum(dh, _CLIP)
    widths = ax2 - ax1 + 1.0
    heights = ay2 - ay1 + 1.0
    ctr_x = ax1 + 0.5 * widths
    ctr_y = ay1 + 0.5 * heights
    pred_ctr_x = dx * widths + ctr_x
    pred_ctr_y = dy * heights + ctr_y
    pred_w = jnp.exp(dw) * widths
    pred_h = jnp.exp(dh) * heights
    x1 = jnp.clip(pred_ctr_x - 0.5 * pred_w, 0.0, _IMG_W - 1.0)
    y1 = jnp.clip(pred_ctr_y - 0.5 * pred_h, 0.0, _IMG_H - 1.0)
    x2 = jnp.clip(pred_ctr_x + 0.5 * pred_w - 1.0, 0.0, _IMG_W - 1.0)
    y2 = jnp.clip(pred_ctr_y + 0.5 * pred_h - 1.0, 0.0, _IMG_H - 1.0)
    return x1, y1, x2, y2


def _rpn_kernel(obj_ref, regt_ref, ancht_ref, reg_ref, anch_ref, out_ref, bnat_ref):
    f32 = jnp.float32

    x1v, y1v, x2v, y2v = _decode_clip(
        regt_ref[0:1, :], regt_ref[1:2, :], regt_ref[2:3, :], regt_ref[3:4, :],
        ancht_ref[0:1, :], ancht_ref[1:2, :], ancht_ref[2:3, :], ancht_ref[3:4, :],
    )
    area_v = (x2v - x1v + 1.0) * (y2v - y1v + 1.0)
    scores = jax.nn.sigmoid(obj_ref[...])

    x1c, y1c, x2c, y2c = _decode_clip(
        reg_ref[:, 0:1], reg_ref[:, 1:2], reg_ref[:, 2:3], reg_ref[:, 3:4],
        anch_ref[:, 0:1], anch_ref[:, 1:2], anch_ref[:, 2:3], anch_ref[:, 3:4],
    )
    bnat_ref[:, 0:1] = x1c
    bnat_ref[:, 1:2] = y1c
    bnat_ref[:, 2:3] = x2c
    bnat_ref[:, 3:4] = y2c

    lane = jax.lax.broadcasted_iota(jnp.int32, (1, _PRE), 1)

    def body(i, keep):
        row = bnat_ref[pl.ds(i, 1), :]
        x1i = row[0, 0]
        y1i = row[0, 1]
        x2i = row[0, 2]
        y2i = row[0, 3]
        iw = jnp.maximum(jnp.minimum(x2v, x2i) - jnp.maximum(x1v, x1i) + 1.0, 0.0)
        ih = jnp.maximum(jnp.minimum(y2v, y2i) - jnp.maximum(y1v, y1i) + 1.0, 0.0)
        inter = iw * ih
        area_i = (x2i - x1i + 1.0) * (y2i - y1i + 1.0)
        iou = inter / (area_i + area_v - inter)
        keep_i = jnp.sum(jnp.where(lane == i, keep, 0.0))
        sup = (iou > _THRESH) & (lane > i) & (keep_i > 0.5)
        return jnp.where(sup, 0.0, keep)

    keep = jax.lax.fori_loop(0, _PRE, body, jnp.ones((1, _PRE), f32))

    tri = (jax.lax.broadcasted_iota(jnp.int32, (_PRE, _PRE), 0)
           <= jax.lax.broadcasted_iota(jnp.int32, (_PRE, _PRE), 1)).astype(jnp.bfloat16)
    csum = jnp.dot(keep.astype(jnp.bfloat16), tri,
                   preferred_element_type=f32)
    nkept = csum[0, _PRE - 1]
    lanef = lane.astype(f32)
    pos = jnp.where(keep > 0.5, csum - 1.0, nkept + lanef - csum)

    rows = jax.lax.broadcasted_iota(jnp.int32, (_POST, _PRE), 0)
    onehot = (rows == pos.astype(jnp.int32)).astype(f32)
    out_ref[:, 0:4] = jnp.dot(onehot, bnat_ref[...], preferred_element_type=f32)
    masked = jnp.where(keep > 0.5, scores, -1.0)
    out_ref[:, 4:5] = jnp.sum(onehot * masked, axis=1, keepdims=True)


@jax.jit
def kernel(objectness, box_regression, anchors):
    top_obj, top_idx = jax.lax.top_k(objectness, _PRE)
    reg = jnp.take(box_regression, top_idx, axis=0)
    anch = jnp.take(anchors, top_idx, axis=0)
    return pl.pallas_call(
        _rpn_kernel,
        out_shape=jax.ShapeDtypeStruct((_POST, 5), jnp.float32),
        scratch_shapes=[pltpu.VMEM((_PRE, 4), jnp.float32)],
    )(top_obj.reshape(1, _PRE), reg.T, anch.T, reg, anch)


# speedup vs baseline: 49.3023x; 1.6242x over previous
import functools

import jax
import jax.numpy as jnp
import numpy as np
from jax.experimental import pallas as pl
from jax.experimental.pallas import tpu as pltpu

_PRE = 2000
_POST = 1000
_THRESH = 0.7
_IMG_W = 800.0
_IMG_H = 800.0
_CLIP = float(np.log(1000.0 / 16.0))


def _decode_clip(dx, dy, dw, dh, ax1, ay1, ax2, ay2):
    dw = jnp.minimum(dw, _CLIP)
    dh = jnp.minimum(dh, _CLIP)
    widths = ax2 - ax1 + 1.0
    heights = ay2 - ay1 + 1.0
    ctr_x = ax1 + 0.5 * widths
    ctr_y = ay1 + 0.5 * heights
    pred_ctr_x = dx * widths + ctr_x
    pred_ctr_y = dy * heights + ctr_y
    pred_w = jnp.exp(dw) * widths
    pred_h = jnp.exp(dh) * heights
    x1 = jnp.clip(pred_ctr_x - 0.5 * pred_w, 0.0, _IMG_W - 1.0)
    y1 = jnp.clip(pred_ctr_y - 0.5 * pred_h, 0.0, _IMG_H - 1.0)
    x2 = jnp.clip(pred_ctr_x + 0.5 * pred_w - 1.0, 0.0, _IMG_W - 1.0)
    y2 = jnp.clip(pred_ctr_y + 0.5 * pred_h - 1.0, 0.0, _IMG_H - 1.0)
    return x1, y1, x2, y2


def _rpn_kernel(obj_ref, regt_ref, ancht_ref, reg_ref, anch_ref, out_ref, bnat_ref):
    f32 = jnp.float32

    x1v, y1v, x2v, y2v = _decode_clip(
        regt_ref[0:1, :], regt_ref[1:2, :], regt_ref[2:3, :], regt_ref[3:4, :],
        ancht_ref[0:1, :], ancht_ref[1:2, :], ancht_ref[2:3, :], ancht_ref[3:4, :],
    )
    area_v = (x2v - x1v + 1.0) * (y2v - y1v + 1.0)
    scores = jax.nn.sigmoid(obj_ref[...])

    x1c, y1c, x2c, y2c = _decode_clip(
        reg_ref[:, 0:1], reg_ref[:, 1:2], reg_ref[:, 2:3], reg_ref[:, 3:4],
        anch_ref[:, 0:1], anch_ref[:, 1:2], anch_ref[:, 2:3], anch_ref[:, 3:4],
    )
    bnat_ref[:, 0:1] = x1c
    bnat_ref[:, 1:2] = y1c
    bnat_ref[:, 2:3] = x2c
    bnat_ref[:, 3:4] = y2c

    lane = jax.lax.broadcasted_iota(jnp.int32, (1, _PRE), 1)

    def body(i, keep):
        row = bnat_ref[pl.ds(i, 1), :]
        x1i = row[0, 0]
        y1i = row[0, 1]
        x2i = row[0, 2]
        y2i = row[0, 3]
        iw = jnp.maximum(jnp.minimum(x2v, x2i) - jnp.maximum(x1v, x1i) + 1.0, 0.0)
        ih = jnp.maximum(jnp.minimum(y2v, y2i) - jnp.maximum(y1v, y1i) + 1.0, 0.0)
        inter = iw * ih
        area_i = (x2i - x1i + 1.0) * (y2i - y1i + 1.0)
        iou = inter / (area_i + area_v - inter)
        keep_i = jnp.sum(jnp.where(lane == i, keep, 0.0))
        sup = (iou > _THRESH) & (lane > i) & (keep_i > 0.5)
        return jnp.where(sup, 0.0, keep)

    def chunk_cond(carry):
        i, keep = carry
        done = jnp.sum(jnp.where(lane < i, keep, 0.0))
        return (i < _PRE) & (done < float(_POST))

    def chunk_body(carry):
        i, keep = carry
        keep = jax.lax.fori_loop(i, jnp.minimum(i + 64, _PRE), body, keep)
        return i + 64, keep

    _, keep = jax.lax.while_loop(
        chunk_cond, chunk_body, (jnp.int32(0), jnp.ones((1, _PRE), f32))
    )

    tri = (jax.lax.broadcasted_iota(jnp.int32, (_PRE, _PRE), 0)
           <= jax.lax.broadcasted_iota(jnp.int32, (_PRE, _PRE), 1)).astype(jnp.bfloat16)
    csum = jnp.dot(keep.astype(jnp.bfloat16), tri,
                   preferred_element_type=f32)
    nkept = csum[0, _PRE - 1]
    lanef = lane.astype(f32)
    pos = jnp.where(keep > 0.5, csum - 1.0, nkept + lanef - csum)

    rows = jax.lax.broadcasted_iota(jnp.int32, (_POST, _PRE), 0)
    onehot = (rows == pos.astype(jnp.int32)).astype(f32)
    out_ref[:, 0:4] = jnp.dot(onehot, bnat_ref[...], preferred_element_type=f32)
    masked = jnp.where(keep > 0.5, scores, -1.0)
    out_ref[:, 4:5] = jnp.sum(onehot * masked, axis=1, keepdims=True)


@jax.jit
def kernel(objectness, box_regression, anchors):
    top_obj, top_idx = jax.lax.top_k(objectness, _PRE)
    reg = jnp.take(box_regression, top_idx, axis=0)
    anch = jnp.take(anchors, top_idx, axis=0)
    return pl.pallas_call(
        _rpn_kernel,
        out_shape=jax.ShapeDtypeStruct((_POST, 5), jnp.float32),
        scratch_shapes=[pltpu.VMEM((_PRE, 4), jnp.float32)],
    )(top_obj.reshape(1, _PRE), reg.T, anch.T, reg, anch)


# speedup vs baseline: 149.0143x; 3.0225x over previous
import functools

import jax
import jax.numpy as jnp
import numpy as np
from jax.experimental import pallas as pl
from jax.experimental.pallas import tpu as pltpu

_PRE = 2000
_POST = 1000
_THRESH = 0.7
_IMG_W = 800.0
_IMG_H = 800.0
_CLIP = float(np.log(1000.0 / 16.0))


def _decode_clip(dx, dy, dw, dh, ax1, ay1, ax2, ay2):
    dw = jnp.minimum(dw, _CLIP)
    dh = jnp.minimum(dh, _CLIP)
    widths = ax2 - ax1 + 1.0
    heights = ay2 - ay1 + 1.0
    ctr_x = ax1 + 0.5 * widths
    ctr_y = ay1 + 0.5 * heights
    pred_ctr_x = dx * widths + ctr_x
    pred_ctr_y = dy * heights + ctr_y
    pred_w = jnp.exp(dw) * widths
    pred_h = jnp.exp(dh) * heights
    x1 = jnp.clip(pred_ctr_x - 0.5 * pred_w, 0.0, _IMG_W - 1.0)
    y1 = jnp.clip(pred_ctr_y - 0.5 * pred_h, 0.0, _IMG_H - 1.0)
    x2 = jnp.clip(pred_ctr_x + 0.5 * pred_w - 1.0, 0.0, _IMG_W - 1.0)
    y2 = jnp.clip(pred_ctr_y + 0.5 * pred_h - 1.0, 0.0, _IMG_H - 1.0)
    return x1, y1, x2, y2


def _rpn_kernel(obj_ref, regt_ref, ancht_ref, reg_ref, anch_ref, out_ref,
                bnat_ref, a_ref):
    f32 = jnp.float32

    x1v, y1v, x2v, y2v = _decode_clip(
        regt_ref[0:1, :], regt_ref[1:2, :], regt_ref[2:3, :], regt_ref[3:4, :],
        ancht_ref[0:1, :], ancht_ref[1:2, :], ancht_ref[2:3, :], ancht_ref[3:4, :],
    )
    area_v = (x2v - x1v + 1.0) * (y2v - y1v + 1.0)
    scores = jax.nn.sigmoid(obj_ref[...])

    x1c, y1c, x2c, y2c = _decode_clip(
        reg_ref[:, 0:1], reg_ref[:, 1:2], reg_ref[:, 2:3], reg_ref[:, 3:4],
        anch_ref[:, 0:1], anch_ref[:, 1:2], anch_ref[:, 2:3], anch_ref[:, 3:4],
    )
    bnat_ref[:, 0:1] = x1c
    bnat_ref[:, 1:2] = y1c
    bnat_ref[:, 2:3] = x2c
    bnat_ref[:, 3:4] = y2c

    lane = jax.lax.broadcasted_iota(jnp.int32, (1, _PRE), 1)

    strip = 250
    for r0 in range(0, _PRE, strip):
        x1s = bnat_ref[r0:r0 + strip, 0:1]
        y1s = bnat_ref[r0:r0 + strip, 1:2]
        x2s = bnat_ref[r0:r0 + strip, 2:3]
        y2s = bnat_ref[r0:r0 + strip, 3:4]
        iw = jnp.maximum(jnp.minimum(x2s, x2v) - jnp.maximum(x1s, x1v) + 1.0, 0.0)
        ih = jnp.maximum(jnp.minimum(y2s, y2v) - jnp.maximum(y1s, y1v) + 1.0, 0.0)
        inter = iw * ih
        area_s = (x2s - x1s + 1.0) * (y2s - y1s + 1.0)
        iou = inter / (area_s + area_v - inter)
        rows_g = jax.lax.broadcasted_iota(jnp.int32, (strip, _PRE), 0) + r0
        a_ref[r0:r0 + strip, :] = ((iou > _THRESH) & (rows_g < lane)).astype(jnp.bfloat16)

    def fix_cond(carry):
        return carry[1]

    def fix_body(carry):
        keep, _ = carry
        v = jnp.dot(keep.astype(jnp.bfloat16), a_ref[...],
                    preferred_element_type=f32)
        new = (v < 0.5).astype(f32)
        return new, jnp.any(new != keep)

    keep, _ = jax.lax.while_loop(
        fix_cond, fix_body, (jnp.ones((1, _PRE), f32), jnp.bool_(True))
    )

    tri = (jax.lax.broadcasted_iota(jnp.int32, (_PRE, _PRE), 0)
           <= jax.lax.broadcasted_iota(jnp.int32, (_PRE, _PRE), 1)).astype(jnp.bfloat16)
    csum = jnp.dot(keep.astype(jnp.bfloat16), tri,
                   preferred_element_type=f32)
    nkept = csum[0, _PRE - 1]
    lanef = lane.astype(f32)
    pos = jnp.where(keep > 0.5, csum - 1.0, nkept + lanef - csum)

    rows = jax.lax.broadcasted_iota(jnp.int32, (_POST, _PRE), 0)
    onehot = (rows == pos.astype(jnp.int32)).astype(f32)
    out_ref[:, 0:4] = jnp.dot(onehot, bnat_ref[...], preferred_element_type=f32)
    masked = jnp.where(keep > 0.5, scores, -1.0)
    out_ref[:, 4:5] = jnp.sum(onehot * masked, axis=1, keepdims=True)


@jax.jit
def kernel(objectness, box_regression, anchors):
    top_obj, top_idx = jax.lax.top_k(objectness, _PRE)
    reg = jnp.take(box_regression, top_idx, axis=0)
    anch = jnp.take(anchors, top_idx, axis=0)
    return pl.pallas_call(
        _rpn_kernel,
        out_shape=jax.ShapeDtypeStruct((_POST, 5), jnp.float32),
        scratch_shapes=[
            pltpu.VMEM((_PRE, 4), jnp.float32),
            pltpu.VMEM((_PRE, _PRE), jnp.bfloat16),
        ],
    )(top_obj.reshape(1, _PRE), reg.T, anch.T, reg, anch)
